# baseline (speedup 1.0000x reference)
import functools

import jax
import jax.numpy as jnp
from jax import lax
from jax.experimental import pallas as pl
from jax.experimental.pallas import tpu as pltpu
from jax.experimental.pallas import tpu_sc as plsc

N = 10000
E = 320000
D_IN = 128
D_H = 128
D_OUT = 64

NC = 2
NS = 16
LANES = 16
NW = NC * NS

EPW = E // NW
K = 40
NCHUNK = EPW // K
NSLOT = 4
N_PAD = 10112
ROWS_PER_TILE = N_PAD // NS


def _splat(vec, j):
    idx = jnp.full((LANES, 1), j, jnp.int32)
    return lax.gather(
        vec, idx,
        lax.GatherDimensionNumbers(
            offset_dims=(), collapsed_slice_dims=(0,), start_index_map=(0,)),
        (1,), mode=lax.GatherScatterMode.PROMISE_IN_BOUNDS)


def _make_spmm(d):
    mesh = plsc.VectorSubcoreMesh(
        core_axis_name="c", subcore_axis_name="s",
        num_cores=NC, num_subcores=NS)

    @functools.partial(
        pl.kernel,
        out_type=jax.ShapeDtypeStruct((NC, N_PAD, d), jnp.float32),
        mesh=mesh,
        compiler_params=pltpu.CompilerParams(use_tc_tiling_on_sc=False),
        scratch_types=[
            pltpu.VMEM((NCHUNK, K), jnp.int32),
            pltpu.VMEM((K, d), jnp.float32),
            pltpu.VMEM((K, d), jnp.float32),
            pltpu.VMEM((K, d), jnp.float32),
            pltpu.VMEM((K, d), jnp.float32),
            pltpu.VMEM((NSLOT, K), jnp.int32),
            pltpu.VMEM((NSLOT, K), jnp.int32),
            pltpu.VMEM((NSLOT, K), jnp.float32),
            pltpu.VMEM_SHARED((N_PAD, d), jnp.float32),
            pltpu.SemaphoreType.DMA,
            pltpu.SemaphoreType.DMA,
            pltpu.SemaphoreType.DMA,
            pltpu.SemaphoreType.DMA,
            pltpu.SemaphoreType.DMA,
            pltpu.SemaphoreType.DMA,
            pltpu.SemaphoreType.DMA,
            pltpu.SemaphoreType.DMA,
        ],
    )
    def spmm(m_hbm, pk_hbm, w_hbm, out_hbm,
             pk_all, r0, r1, r2, r3, src_b, dst_b, w_b, accum,
             g0, g1, g2, g3, s0, s1, s2, s3):
        c = lax.axis_index("c")
        s = lax.axis_index("s")
        wid = c * NS + s
        rows = (r0, r1, r2, r3)
        gsem = (g0, g1, g2, g3)
        ssem = (s0, s1, s2, s3)

        pltpu.sync_copy(pk_hbm.at[wid], pk_all)

        zeros16 = jnp.zeros((LANES,), jnp.float32)

        def zrow(r, _):
            for i in range(d // LANES):
                r0[r, pl.ds(i * LANES, LANES)] = zeros16
            return 0

        lax.fori_loop(0, K, zrow, 0)
        for r in range(ROWS_PER_TILE // K):
            pltpu.sync_copy(
                r0, accum.at[pl.ds(s * ROWS_PER_TILE + r * K, K)])
        zrem = ROWS_PER_TILE % K
        if zrem:
            pltpu.sync_copy(
                r0.at[pl.ds(0, zrem)],
                accum.at[pl.ds(s * ROWS_PER_TILE + ROWS_PER_TILE - zrem,
                               zrem)])
        plsc.subcore_barrier()

        _windows = [(g * LANES, 0, LANES) for g in range(K // LANES)]
        if K % LANES:
            _windows.append((K - LANES, LANES - (K % LANES), K % LANES))

        def unpack_idx(ci, slot):
            for off, lo, cnt in _windows:
                p = pk_all[ci, pl.ds(off, LANES)]
                src_b[slot, pl.ds(off, LANES)] = p & jnp.int32(0xFFFF)
                dst_b[slot, pl.ds(off, LANES)] = lax.shift_right_logical(
                    p, jnp.int32(16))

        def gather_start(ci, slot):
            unpack_idx(ci, slot)
            pltpu.async_copy(m_hbm.at[src_b.at[slot]], rows[slot],
                             gsem[slot])
            pltpu.async_copy(w_hbm.at[wid, ci], w_b.at[slot], gsem[slot])

        def gather_wait(slot):
            pltpu.make_async_copy(
                m_hbm.at[pl.ds(0, K)], rows[slot], gsem[slot]).wait()
            pltpu.make_async_copy(
                w_hbm.at[0, 0], w_b.at[slot], gsem[slot]).wait()

        def scatter_start(ci, slot):
            pltpu.async_copy(rows[slot], accum.at[dst_b.at[slot]],
                             ssem[slot], add=True)

        def scatter_wait(slot):
            pltpu.make_async_copy(
                m_hbm.at[pl.ds(0, K)], rows[slot], ssem[slot]).wait()

        def scale(ci, slot):
            rv = rows[slot]
            for off, lo, cnt in _windows:
                w16 = w_b[slot, pl.ds(off, LANES)]
                for j in range(lo, LANES):
                    wj = _splat(w16, j)
                    row = off + j
                    for i in range(d // LANES):
                        sl = (row, pl.ds(i * LANES, LANES))
                        rv[sl] = rv[sl] * wj

        gather_start(0, 0)
        gather_start(1, 1)
        for ci in (0, 1):
            gather_wait(ci)
            scale(ci, ci)
            scatter_start(ci, ci)
            gather_start(ci + 2, ci + 2)

        def quad_body(q, _):
            for b in range(4):
                slot = (2 + b) % 4
                ci = 2 + q * 4 + b
                gather_wait(slot)
                scale(ci, slot)
                scatter_start(ci, slot)
                pf = ci + 2
                pf_slot = (slot + 2) % 4

                @pl.when(pf < NCHUNK)
                def _():
                    scatter_wait(pf_slot)
                    gather_start(pf, pf_slot)
            return 0

        lax.fori_loop(0, (NCHUNK - 2) // 4, quad_body, 0)
        for slot in range(NSLOT):
            scatter_wait(slot)
        plsc.subcore_barrier()

        pltpu.sync_copy(
            accum.at[pl.ds(s * ROWS_PER_TILE, ROWS_PER_TILE)],
            out_hbm.at[c, pl.ds(s * ROWS_PER_TILE, ROWS_PER_TILE)])

    return spmm


_spmm_h = _make_spmm(D_H)


def _mm_body(x_ref, w_ref, o_ref):
    o_ref[...] = jnp.dot(x_ref[...], w_ref[...],
                         preferred_element_type=jnp.float32)


def _relu_add_body(a_ref, b_ref, o_ref):
    o_ref[...] = jnp.maximum(a_ref[...] + b_ref[...], 0.0)


def _add_mm_logsoftmax_body(a_ref, b_ref, w_ref, o_ref):
    x = jnp.dot(a_ref[:N, :] + b_ref[:N, :], w_ref[...],
                preferred_element_type=jnp.float32)
    m = jnp.max(x, axis=1, keepdims=True)
    xs = x - m
    o_ref[...] = xs - jnp.log(jnp.sum(jnp.exp(xs), axis=1, keepdims=True))


def kernel(X, edge_index, edge_weight, W1, W2):
    pk = (edge_index[0] | (edge_index[1] << 16)).reshape(NW, NCHUNK, K)
    w = edge_weight.reshape(NW, NCHUNK, K)

    xw1 = pl.pallas_call(
        _mm_body,
        out_shape=jax.ShapeDtypeStruct((N, D_H), jnp.float32),
    )(X, W1)

    p1 = _spmm_h(xw1, pk, w)

    h = pl.pallas_call(
        _relu_add_body,
        out_shape=jax.ShapeDtypeStruct((N_PAD, D_H), jnp.float32),
    )(p1[0], p1[1])

    p2 = _spmm_h(h, pk, w)

    out = pl.pallas_call(
        _add_mm_logsoftmax_body,
        out_shape=jax.ShapeDtypeStruct((N, D_OUT), jnp.float32),
    )(p2[0], p2[1], W2)

    return out


# speedup vs baseline: 1.0192x; 1.0192x over previous
import functools

import jax
import jax.numpy as jnp
from jax import lax
from jax.experimental import pallas as pl
from jax.experimental.pallas import tpu as pltpu
from jax.experimental.pallas import tpu_sc as plsc

N = 10000
E = 320000
D_IN = 128
D_H = 128
D_OUT = 64

NC = 2
NS = 16
LANES = 16
NW = NC * NS

EPW = E // NW
K = 40
NCHUNK = EPW // K
NSLOT = 4
N_PAD = 10112
ROWS_PER_TILE = N_PAD // NS


def _splat(vec, j):
    idx = jnp.full((LANES, 1), j, jnp.int32)
    return lax.gather(
        vec, idx,
        lax.GatherDimensionNumbers(
            offset_dims=(), collapsed_slice_dims=(0,), start_index_map=(0,)),
        (1,), mode=lax.GatherScatterMode.PROMISE_IN_BOUNDS)


def _make_spmm(d):
    mesh = plsc.VectorSubcoreMesh(
        core_axis_name="c", subcore_axis_name="s",
        num_cores=NC, num_subcores=NS)

    @functools.partial(
        pl.kernel,
        out_type=jax.ShapeDtypeStruct((NC, N_PAD, d), jnp.float32),
        mesh=mesh,
        scratch_types=[
            pltpu.VMEM((128, K), jnp.int32),
            pltpu.VMEM((K, d), jnp.float32),
            pltpu.VMEM((K, d), jnp.float32),
            pltpu.VMEM((K, d), jnp.float32),
            pltpu.VMEM((K, d), jnp.float32),
            pltpu.VMEM((NSLOT, K), jnp.int32),
            pltpu.VMEM((NSLOT, K), jnp.int32),
            pltpu.VMEM((NSLOT, K), jnp.float32),
            pltpu.VMEM_SHARED((N_PAD, d), jnp.float32),
            pltpu.SemaphoreType.DMA,
            pltpu.SemaphoreType.DMA,
            pltpu.SemaphoreType.DMA,
            pltpu.SemaphoreType.DMA,
            pltpu.SemaphoreType.DMA,
            pltpu.SemaphoreType.DMA,
            pltpu.SemaphoreType.DMA,
            pltpu.SemaphoreType.DMA,
        ],
    )
    def spmm(m_hbm, pk_hbm, w_hbm, out_hbm,
             pk_all, r0, r1, r2, r3, src_b, dst_b, w_b, accum,
             g0, g1, g2, g3, s0, s1, s2, s3):
        c = lax.axis_index("c")
        s = lax.axis_index("s")
        wid = c * NS + s
        rows = (r0, r1, r2, r3)
        gsem = (g0, g1, g2, g3)
        ssem = (s0, s1, s2, s3)

        pltpu.sync_copy(pk_hbm.at[wid, pl.ds(0, 128)], pk_all)

        zeros16 = jnp.zeros((LANES,), jnp.float32)

        def zrow(r, _):
            for i in range(d // LANES):
                r0[r, pl.ds(i * LANES, LANES)] = zeros16
            return 0

        lax.fori_loop(0, K, zrow, 0)
        for r in range(ROWS_PER_TILE // K):
            pltpu.sync_copy(
                r0, accum.at[pl.ds(s * ROWS_PER_TILE + r * K, K)])
        zrem = ROWS_PER_TILE % K
        if zrem:
            pltpu.sync_copy(
                r0.at[pl.ds(0, zrem)],
                accum.at[pl.ds(s * ROWS_PER_TILE + ROWS_PER_TILE - zrem,
                               zrem)])
        plsc.subcore_barrier()

        _windows = [(g * LANES, 0, LANES) for g in range(K // LANES)]
        if K % LANES:
            _windows.append((K - LANES, LANES - (K % LANES), K % LANES))

        def unpack_idx(pkrow, slot):
            for off, lo, cnt in _windows:
                p = pk_all[pkrow, pl.ds(off, LANES)]
                src_b[slot, pl.ds(off, LANES)] = p & jnp.int32(0xFFFF)
                dst_b[slot, pl.ds(off, LANES)] = lax.shift_right_logical(
                    p, jnp.int32(16))

        def gather_start(ci, slot, half=0):
            unpack_idx(ci - half * 128, slot)
            pltpu.async_copy(m_hbm.at[src_b.at[slot]], rows[slot],
                             gsem[slot])
            pltpu.async_copy(w_hbm.at[wid, ci], w_b.at[slot], gsem[slot])

        def gather_wait(slot):
            pltpu.make_async_copy(
                m_hbm.at[pl.ds(0, K)], rows[slot], gsem[slot]).wait()
            pltpu.make_async_copy(
                w_hbm.at[0, 0], w_b.at[slot], gsem[slot]).wait()

        def scatter_start(ci, slot):
            pltpu.async_copy(rows[slot], accum.at[dst_b.at[slot]],
                             ssem[slot], add=True)

        def scatter_wait(slot):
            pltpu.make_async_copy(
                m_hbm.at[pl.ds(0, K)], rows[slot], ssem[slot]).wait()

        def scale(ci, slot):
            rv = rows[slot]
            for off, lo, cnt in _windows:
                w16 = w_b[slot, pl.ds(off, LANES)]
                for j in range(lo, LANES):
                    wj = _splat(w16, j)
                    row = off + j
                    for i in range(d // LANES):
                        sl = (row, pl.ds(i * LANES, LANES))
                        rv[sl] = rv[sl] * wj

        gather_start(0, 0)
        gather_start(1, 1)
        for ci in (0, 1):
            gather_wait(ci)
            scale(ci, ci)
            scatter_start(ci, ci)
            gather_start(ci + 2, ci + 2)

        def quad_a(q, _):
            for b in range(4):
                slot = (2 + b) % 4
                ci = 2 + q * 4 + b
                gather_wait(slot)
                scale(ci, slot)
                scatter_start(ci, slot)
                pf_slot = (slot + 2) % 4
                scatter_wait(pf_slot)
                gather_start(ci + 2, pf_slot, half=0)
            return 0

        lax.fori_loop(0, 31, quad_a, 0)
        pltpu.sync_copy(pk_hbm.at[wid, pl.ds(128, NCHUNK - 128)],
                        pk_all.at[pl.ds(0, NCHUNK - 128)])

        def quad_b(q, _):
            for b in range(4):
                slot = (2 + b) % 4
                ci = 126 + q * 4 + b
                gather_wait(slot)
                scale(ci, slot)
                scatter_start(ci, slot)
                pf = ci + 2
                pf_slot = (slot + 2) % 4

                @pl.when(pf < NCHUNK)
                def _():
                    scatter_wait(pf_slot)
                    gather_start(pf, pf_slot, half=1)
            return 0

        lax.fori_loop(0, 31, quad_b, 0)
        for slot in range(NSLOT):
            scatter_wait(slot)
        plsc.subcore_barrier()

        pltpu.sync_copy(
            accum.at[pl.ds(s * ROWS_PER_TILE, ROWS_PER_TILE)],
            out_hbm.at[c, pl.ds(s * ROWS_PER_TILE, ROWS_PER_TILE)])

    return spmm


_spmm_h = _make_spmm(D_H)


def _mm_body(x_ref, w_ref, o_ref):
    o_ref[...] = jnp.dot(x_ref[...], w_ref[...],
                         preferred_element_type=jnp.float32)


def _relu_add_body(a_ref, b_ref, o_ref):
    o_ref[...] = jnp.maximum(a_ref[...] + b_ref[...], 0.0)


def _add_mm_logsoftmax_body(a_ref, b_ref, w_ref, o_ref):
    x = jnp.dot(a_ref[:N, :] + b_ref[:N, :], w_ref[...],
                preferred_element_type=jnp.float32)
    m = jnp.max(x, axis=1, keepdims=True)
    xs = x - m
    o_ref[...] = xs - jnp.log(jnp.sum(jnp.exp(xs), axis=1, keepdims=True))


def kernel(X, edge_index, edge_weight, W1, W2):
    pk = (edge_index[0] | (edge_index[1] << 16)).reshape(NW, NCHUNK, K)
    w = edge_weight.reshape(NW, NCHUNK, K)

    xw1 = pl.pallas_call(
        _mm_body,
        out_shape=jax.ShapeDtypeStruct((N, D_H), jnp.float32),
    )(X, W1)

    p1 = _spmm_h(xw1, pk, w)

    h = pl.pallas_call(
        _relu_add_body,
        out_shape=jax.ShapeDtypeStruct((N_PAD, D_H), jnp.float32),
    )(p1[0], p1[1])

    p2 = _spmm_h(h, pk, w)

    out = pl.pallas_call(
        _add_mm_logsoftmax_body,
        out_shape=jax.ShapeDtypeStruct((N, D_OUT), jnp.float32),
    )(p2[0], p2[1], W2)

    return out


# speedup vs baseline: 1.3266x; 1.3016x over previous
import functools

import jax
import jax.numpy as jnp
from jax import lax
from jax.experimental import pallas as pl
from jax.experimental.pallas import tpu as pltpu
from jax.experimental.pallas import tpu_sc as plsc

N = 10000
E = 320000
D_IN = 128
D_H = 128
D_OUT = 64

NC = 2
NS = 16
LANES = 16
NW = NC * NS

EPW = E // NW
K1 = 40
K2 = 80
NSLOT = 3
N_PAD = 10112
ROWS_PER_TILE = N_PAD // NS


def _splat(vec, j):
    idx = jnp.full((LANES, 1), j, jnp.int32)
    return lax.gather(
        vec, idx,
        lax.GatherDimensionNumbers(
            offset_dims=(), collapsed_slice_dims=(0,), start_index_map=(0,)),
        (1,), mode=lax.GatherScatterMode.PROMISE_IN_BOUNDS)


def _make_spmm(d, K, untiled=False):
    NCHUNK = EPW // K
    mesh = plsc.VectorSubcoreMesh(
        core_axis_name="c", subcore_axis_name="s",
        num_cores=NC, num_subcores=NS)
    params = (pltpu.CompilerParams(use_tc_tiling_on_sc=False)
              if untiled else None)

    @functools.partial(
        pl.kernel,
        out_type=jax.ShapeDtypeStruct((NC, N_PAD, d), jnp.float32),
        mesh=mesh,
        compiler_params=params,
        scratch_types=[
            pltpu.VMEM((NCHUNK, K), jnp.int32),
            pltpu.VMEM((K, d), jnp.float32),
            pltpu.VMEM((K, d), jnp.float32),
            pltpu.VMEM((K, d), jnp.float32),
            pltpu.VMEM((NSLOT, K), jnp.int32),
            pltpu.VMEM((NSLOT, K), jnp.int32),
            pltpu.VMEM((NSLOT, K), jnp.float32),
            pltpu.VMEM_SHARED((N_PAD, d), jnp.float32),
            pltpu.SemaphoreType.DMA,
            pltpu.SemaphoreType.DMA,
            pltpu.SemaphoreType.DMA,
            pltpu.SemaphoreType.DMA,
            pltpu.SemaphoreType.DMA,
            pltpu.SemaphoreType.DMA,
        ],
    )
    def spmm(m_hbm, pk_hbm, w_hbm, out_hbm,
             pk_all, r0, r1, r2, src_b, dst_b, w_b, accum,
             g0, g1, g2, s0, s1, s2):
        c = lax.axis_index("c")
        s = lax.axis_index("s")
        wid = c * NS + s
        rows = (r0, r1, r2)
        gsem = (g0, g1, g2)
        ssem = (s0, s1, s2)

        pltpu.sync_copy(pk_hbm.at[wid], pk_all)

        zeros16 = jnp.zeros((LANES,), jnp.float32)

        def zrow(r, _):
            for i in range(d // LANES):
                r0[r, pl.ds(i * LANES, LANES)] = zeros16
            return 0

        lax.fori_loop(0, K, zrow, 0)
        for r in range(ROWS_PER_TILE // K):
            pltpu.sync_copy(
                r0, accum.at[pl.ds(s * ROWS_PER_TILE + r * K, K)])
        zrem = ROWS_PER_TILE % K
        if zrem:
            pltpu.sync_copy(
                r0.at[pl.ds(0, zrem)],
                accum.at[pl.ds(s * ROWS_PER_TILE + ROWS_PER_TILE - zrem,
                               zrem)])
        plsc.subcore_barrier()

        _windows = [(g * LANES, 0, LANES) for g in range(K // LANES)]
        if K % LANES:
            _windows.append((K - LANES, LANES - (K % LANES), K % LANES))

        def unpack_idx(ci, slot):
            for off, lo, cnt in _windows:
                p = pk_all[ci, pl.ds(off, LANES)]
                src_b[slot, pl.ds(off, LANES)] = p & jnp.int32(0xFFFF)
                dst_b[slot, pl.ds(off, LANES)] = lax.shift_right_logical(
                    p, jnp.int32(16))

        def gather_start(ci, slot):
            unpack_idx(ci, slot)
            pltpu.async_copy(m_hbm.at[src_b.at[slot]], rows[slot],
                             gsem[slot])
            pltpu.async_copy(w_hbm.at[wid, ci], w_b.at[slot], gsem[slot])

        def gather_wait(slot):
            pltpu.make_async_copy(
                m_hbm.at[pl.ds(0, K)], rows[slot], gsem[slot]).wait()
            pltpu.make_async_copy(
                w_hbm.at[0, 0], w_b.at[slot], gsem[slot]).wait()

        def scatter_start(ci, slot):
            pltpu.async_copy(rows[slot], accum.at[dst_b.at[slot]],
                             ssem[slot], add=True)

        def scatter_wait(slot):
            pltpu.make_async_copy(
                m_hbm.at[pl.ds(0, K)], rows[slot], ssem[slot]).wait()

        def scale(ci, slot):
            rv = rows[slot]
            for off, lo, cnt in _windows:
                w16 = w_b[slot, pl.ds(off, LANES)]
                for j in range(lo, LANES):
                    wj = _splat(w16, j)
                    row = off + j
                    for i in range(d // LANES):
                        sl = (row, pl.ds(i * LANES, LANES))
                        rv[sl] = rv[sl] * wj

        gather_start(0, 0)
        gather_start(1, 1)
        gather_wait(0)
        scale(0, 0)
        scatter_start(0, 0)
        gather_start(2, 2)
        gather_wait(1)
        scale(1, 1)
        scatter_start(1, 1)
        scatter_wait(0)
        gather_start(3, 0)

        def triple_body(q, _):
            for b in range(3):
                slot = (2 + b) % 3
                ci = 2 + q * 3 + b
                gather_wait(slot)
                scale(ci, slot)
                scatter_start(ci, slot)
                pf_slot = (slot + 2) % 3
                scatter_wait(pf_slot)
                gather_start(ci + 2, pf_slot)
            return 0

        lax.fori_loop(0, (NCHUNK - 4) // 3, triple_body, 0)
        for ci in range(2 + 3 * ((NCHUNK - 4) // 3), NCHUNK):
            slot = ci % 3
            gather_wait(slot)
            scale(ci, slot)
            scatter_start(ci, slot)
            if ci + 2 < NCHUNK:
                pf_slot = (slot + 2) % 3
                scatter_wait(pf_slot)
                gather_start(ci + 2, pf_slot)
        for slot in range(NSLOT):
            scatter_wait(slot)
        plsc.subcore_barrier()

        pltpu.sync_copy(
            accum.at[pl.ds(s * ROWS_PER_TILE, ROWS_PER_TILE)],
            out_hbm.at[c, pl.ds(s * ROWS_PER_TILE, ROWS_PER_TILE)])

    return spmm


_spmm_h = _make_spmm(D_H, K1)
_spmm_out = _make_spmm(D_OUT, K2, untiled=True)


def _mm_body(x_ref, w_ref, o_ref):
    o_ref[...] = jnp.dot(x_ref[...], w_ref[...],
                         preferred_element_type=jnp.float32)


def _relu_add_mm_body(a_ref, b_ref, w_ref, o_ref):
    h = jnp.maximum(a_ref[...] + b_ref[...], 0.0)
    o_ref[...] = jnp.dot(h, w_ref[...], preferred_element_type=jnp.float32)


def _add_logsoftmax_body(a_ref, b_ref, o_ref):
    x = a_ref[:N, :] + b_ref[:N, :]
    m = jnp.max(x, axis=1, keepdims=True)
    xs = x - m
    o_ref[...] = xs - jnp.log(jnp.sum(jnp.exp(xs), axis=1, keepdims=True))


def kernel(X, edge_index, edge_weight, W1, W2):
    pk = edge_index[0] | (edge_index[1] << 16)
    pk1 = pk.reshape(NW, EPW // K1, K1)
    w1 = edge_weight.reshape(NW, EPW // K1, K1)
    pk2 = pk.reshape(NW, EPW // K2, K2)
    w2 = edge_weight.reshape(NW, EPW // K2, K2)

    xw1 = pl.pallas_call(
        _mm_body,
        out_shape=jax.ShapeDtypeStruct((N, D_H), jnp.float32),
    )(X, W1)

    p1 = _spmm_h(xw1, pk1, w1)

    hw2 = pl.pallas_call(
        _relu_add_mm_body,
        out_shape=jax.ShapeDtypeStruct((N_PAD, D_OUT), jnp.float32),
    )(p1[0], p1[1], W2)

    p2 = _spmm_out(hw2, pk2, w2)

    out = pl.pallas_call(
        _add_logsoftmax_body,
        out_shape=jax.ShapeDtypeStruct((N, D_OUT), jnp.float32),
    )(p2[0], p2[1])

    return out
